# baseline (speedup 1.0000x reference)
import functools

import jax
import jax.numpy as jnp
from jax import lax
from jax.experimental import pallas as pl
from jax.experimental.pallas import tpu as pltpu
from jax.experimental.pallas import tpu_sc as plsc

U = 10000
N = 10000
E = 320000
D = 128

NC = 2
NS = 16
NW = NC * NS
EPW = E // NW
CB = 64
NCHUNK = E // CB
NQMAX = (NCHUNK + NW - 1) // NW
ZCH = U // CB
ZTAIL = U - ZCH * CB
CNTB = 200
OCB = 200
NOCH = U // OCB

DEPTH = 3
NGROUP = (NQMAX + DEPTH - 1) // DEPTH


def _sc_pass_body(table, gidx, sidx, edge, acc_out, cnt_out, *refs):
    gidx_v = refs[0:DEPTH]
    sidx_v = refs[DEPTH:2 * DEPTH]
    edge_v = refs[2 * DEPTH:3 * DEPTH]
    rows_v = refs[3 * DEPTH:4 * DEPTH]
    zrow_v, ones_v, cnt_v = refs[4 * DEPTH:4 * DEPTH + 3]
    loadsem = refs[4 * DEPTH + 3:5 * DEPTH + 3]
    gathsem = refs[5 * DEPTH + 3:6 * DEPTH + 3]
    scatsem = refs[6 * DEPTH + 3:7 * DEPTH + 3]
    acc_sp, cnt_sp = refs[7 * DEPTH + 3:7 * DEPTH + 5]

    c = lax.axis_index("c")
    s = lax.axis_index("s")
    wid = s * NC + c
    if NCHUNK % NW:
        nq = jnp.where(wid < NCHUNK % NW, NQMAX, NQMAX - 1).astype(jnp.int32)
    else:
        nq = jnp.int32(NCHUNK // NW)

    def _fill_small(i, carry):
        zrow_v[pl.ds(i * 16, 16)] = jnp.zeros((16,), jnp.float32)
        ones_v[pl.ds(i * 16, 16)] = jnp.full((16,), 1.0, jnp.float32)
        return carry
    lax.fori_loop(0, CB // 16, _fill_small, 0)

    def _fill_zbuf(r, carry):
        for j in range(D // 16):
            edge_v[0][r, pl.ds(j * 16, 16)] = jnp.zeros((16,), jnp.float32)
        return carry
    lax.fori_loop(0, CB, _fill_zbuf, 0)

    def _zero(k, carry):
        ch = s + k * NS

        @pl.when(ch < ZCH)
        def _():
            pltpu.sync_copy(edge_v[0], acc_sp.at[pl.ds(ch * CB, CB)])
            pltpu.sync_copy(zrow_v, cnt_sp.at[pl.ds(ch * CB, CB)])
        return carry
    lax.fori_loop(0, (ZCH + NS - 1) // NS, _zero, 0)

    @pl.when(s == 0)
    def _():
        pltpu.sync_copy(edge_v[0].at[pl.ds(0, ZTAIL)],
                        acc_sp.at[pl.ds(ZCH * CB, ZTAIL)])
        pltpu.sync_copy(zrow_v.at[pl.ds(0, ZTAIL)],
                        cnt_sp.at[pl.ds(ZCH * CB, ZTAIL)])

    plsc.subcore_barrier()

    def _start_loads(q, j):
        base = (q * NW + wid) * CB
        pltpu.async_copy(gidx.at[pl.ds(base, CB)], gidx_v[j], loadsem[j])
        pltpu.async_copy(sidx.at[pl.ds(base, CB)], sidx_v[j], loadsem[j])
        pltpu.async_copy(edge.at[pl.ds(base, CB)], edge_v[j], loadsem[j])

    def _wait_loads(j):
        pltpu.make_async_copy(gidx.at[pl.ds(0, CB)], gidx_v[j], loadsem[j]).wait()
        pltpu.make_async_copy(sidx.at[pl.ds(0, CB)], sidx_v[j], loadsem[j]).wait()
        pltpu.make_async_copy(edge.at[pl.ds(0, CB)], edge_v[j], loadsem[j]).wait()

    def _start_gather(j):
        pltpu.async_copy(table.at[gidx_v[j]], rows_v[j], gathsem[j])

    def _wait_gather(j):
        pltpu.make_async_copy(table.at[gidx_v[j]], rows_v[j], gathsem[j]).wait()

    def _start_scatter(j):
        pltpu.async_copy(edge_v[j], acc_sp.at[sidx_v[j]], scatsem[j], add=True)
        pltpu.async_copy(ones_v, cnt_sp.at[sidx_v[j]], scatsem[j], add=True)

    def _wait_scatter(j):
        pltpu.make_async_copy(edge_v[j], acc_sp.at[sidx_v[j]], scatsem[j]).wait()
        pltpu.make_async_copy(ones_v, cnt_sp.at[sidx_v[j]], scatsem[j]).wait()

    for q0 in range(DEPTH - 1):
        _start_loads(q0, q0)
    _wait_loads(0)
    _start_gather(0)

    def _group(g, carry):
        for j in range(DEPTH):
            q = g * DEPTH + j
            j1 = (j + 1) % DEPTH
            jp = (j - 1) % DEPTH

            @pl.when(q < nq)
            def _():
                @pl.when(q + 1 < nq)
                def _():
                    _wait_loads(j1)
                    _start_gather(j1)

                @pl.when(q >= 1)
                def _():
                    _wait_scatter(jp)

                @pl.when(q + DEPTH - 1 < nq)
                def _():
                    _start_loads(q + DEPTH - 1, jp)

                _wait_gather(j)

                shift16 = jnp.full((16,), 16, jnp.int32)
                mask_hi = jnp.full((16,), -65536, jnp.int32)

                def _mul(r, carry2):
                    for g in range(D // 32):
                        w = rows_v[j][r, pl.ds(16 * g, 16)]
                        lo = lax.bitcast_convert_type(w << shift16, jnp.float32)
                        hi = lax.bitcast_convert_type(w & mask_hi, jnp.float32)
                        sl_lo = pl.ds(32 * g, 16)
                        sl_hi = pl.ds(32 * g + 16, 16)
                        edge_v[j][r, sl_lo] = lo * edge_v[j][r, sl_lo]
                        edge_v[j][r, sl_hi] = hi * edge_v[j][r, sl_hi]
                    return carry2
                lax.fori_loop(0, CB, _mul, 0)

                _start_scatter(j)
        return carry
    lax.fori_loop(0, NGROUP, _group, 0)

    for jd in range(DEPTH):
        @pl.when(lax.rem(nq - 1, jnp.int32(DEPTH)) == jd)
        def _(jd=jd):
            _wait_scatter(jd)

    plsc.subcore_barrier()

    def _copy_out(k, carry):
        ch = s + k * NS

        @pl.when(ch < NOCH)
        def _():
            pltpu.sync_copy(acc_sp.at[pl.ds(ch * OCB, OCB)],
                            acc_out.at[pl.ds(c * U + ch * OCB, OCB)])
        return carry
    lax.fori_loop(0, (NOCH + NS - 1) // NS, _copy_out, 0)

    def _cnt_out(k, carry):
        ch = s + k * NS

        @pl.when(ch < U // CNTB)
        def _():
            pltpu.sync_copy(cnt_sp.at[pl.ds(ch * CNTB, CNTB)], cnt_v)
            pltpu.sync_copy(cnt_v, cnt_out.at[pl.ds(c * U + ch * CNTB, CNTB)])
        return carry
    lax.fori_loop(0, (U // CNTB + NS - 1) // NS, _cnt_out, 0)


_sc_pass = functools.partial(
    pl.kernel,
    out_type=(jax.ShapeDtypeStruct((NC * U, D), jnp.float32),
              jax.ShapeDtypeStruct((NC * U,), jnp.float32)),
    mesh=plsc.VectorSubcoreMesh(core_axis_name="c", subcore_axis_name="s",
                                num_cores=NC, num_subcores=NS),
    compiler_params=pltpu.CompilerParams(use_tc_tiling_on_sc=False),
    scratch_types=(
        [pltpu.VMEM((CB,), jnp.int32) for _ in range(DEPTH)]
        + [pltpu.VMEM((CB,), jnp.int32) for _ in range(DEPTH)]
        + [pltpu.VMEM((CB, D), jnp.float32) for _ in range(DEPTH)]
        + [pltpu.VMEM((CB, D // 2), jnp.int32) for _ in range(DEPTH)]
        + [pltpu.VMEM((CB,), jnp.float32),
           pltpu.VMEM((CB,), jnp.float32),
           pltpu.VMEM((CNTB,), jnp.float32)]
        + [pltpu.SemaphoreType.DMA for _ in range(3 * DEPTH)]
        + [pltpu.VMEM_SHARED((U, D), jnp.float32),
           pltpu.VMEM_SHARED((U,), jnp.float32)]
    ),
)(_sc_pass_body)


BU = 1000


def _lstm_body(unew_ref, hn_ref, cs_ref, wih_ref, whh_ref,
               b_ref, hn_out_ref, cs_out_ref):
    gates = (jnp.dot(unew_ref[...], wih_ref[...],
                     preferred_element_type=jnp.float32)
             + jnp.dot(hn_ref[...], whh_ref[...],
                       preferred_element_type=jnp.float32)
             + b_ref[...])
    i = jax.nn.sigmoid(gates[:, 0:D])
    f = jax.nn.sigmoid(gates[:, D:2 * D])
    g = jnp.tanh(gates[:, 2 * D:3 * D])
    o = jax.nn.sigmoid(gates[:, 3 * D:4 * D])
    cs = f * cs_ref[...] + i * g
    cs_out_ref[...] = cs
    hn_out_ref[...] = o * jnp.tanh(cs)


def _norm_body(acc_ref, cnt_ref, nf_ref, out_ref):
    a = acc_ref[0] + acc_ref[1]
    cnt = cnt_ref[0] + cnt_ref[1]
    out_ref[...] = a / jnp.maximum(cnt, 1.0) + nf_ref[...]


def _interleave_bf16(x):
    r = x.shape[0]
    y = (x.reshape(r, D // 32, 2, 16).swapaxes(2, 3)
         .reshape(r, D // 2, 2).astype(jnp.bfloat16))
    return lax.bitcast_convert_type(y, jnp.int32)


def kernel(user_feat, news_feat, edge_src_user, edge_dst_news, edge_feat,
           edge_feat_rev, prev_hn, prev_cs, W_ih, W_hh, b_ih, b_hh):
    gidx1 = edge_dst_news.astype(jnp.int32)
    sidx1 = edge_src_user.astype(jnp.int32)

    acc1, cnt1 = _sc_pass(_interleave_bf16(news_feat), gidx1, sidx1,
                          edge_feat_rev)
    acc1 = acc1.reshape(NC, U, D)
    cnt1 = cnt1.reshape(NC, U, 1)

    wihT = W_ih.T
    whhT = W_hh.T
    b = (b_ih + b_hh).reshape(1, 4 * D)

    user_new = pl.pallas_call(
        _norm_body,
        grid=(U // BU,),
        in_specs=[
            pl.BlockSpec((NC, BU, D), lambda i: (0, i, 0)),
            pl.BlockSpec((NC, BU, 1), lambda i: (0, i, 0)),
            pl.BlockSpec((BU, D), lambda i: (i, 0)),
        ],
        out_specs=pl.BlockSpec((BU, D), lambda i: (i, 0)),
        out_shape=jax.ShapeDtypeStruct((U, D), jnp.float32),
    )(acc1, cnt1, user_feat)

    user_hn, user_cs = pl.pallas_call(
        _lstm_body,
        grid=(U // BU,),
        in_specs=[
            pl.BlockSpec((BU, D), lambda i: (i, 0)),
            pl.BlockSpec((BU, D), lambda i: (i, 0)),
            pl.BlockSpec((BU, D), lambda i: (i, 0)),
            pl.BlockSpec((D, 4 * D), lambda i: (0, 0)),
            pl.BlockSpec((D, 4 * D), lambda i: (0, 0)),
            pl.BlockSpec((1, 4 * D), lambda i: (0, 0)),
        ],
        out_specs=[
            pl.BlockSpec((BU, D), lambda i: (i, 0)),
            pl.BlockSpec((BU, D), lambda i: (i, 0)),
        ],
        out_shape=[
            jax.ShapeDtypeStruct((U, D), jnp.float32),
            jax.ShapeDtypeStruct((U, D), jnp.float32),
        ],
    )(user_new, prev_hn, prev_cs, wihT, whhT, b)

    acc2, cnt2 = _sc_pass(_interleave_bf16(user_new), sidx1, gidx1, edge_feat)
    acc2 = acc2.reshape(NC, N, D)
    cnt2 = cnt2.reshape(NC, N, 1)

    news_new = pl.pallas_call(
        _norm_body,
        grid=(N // BU,),
        in_specs=[
            pl.BlockSpec((NC, BU, D), lambda i: (0, i, 0)),
            pl.BlockSpec((NC, BU, 1), lambda i: (0, i, 0)),
            pl.BlockSpec((BU, D), lambda i: (i, 0)),
        ],
        out_specs=pl.BlockSpec((BU, D), lambda i: (i, 0)),
        out_shape=jax.ShapeDtypeStruct((N, D), jnp.float32),
    )(acc2, cnt2, news_feat)

    return user_hn, user_cs, news_new


# speedup vs baseline: 1.4170x; 1.4170x over previous
import functools

import jax
import jax.numpy as jnp
from jax import lax
from jax.experimental import pallas as pl
from jax.experimental.pallas import tpu as pltpu
from jax.experimental.pallas import tpu_sc as plsc

U = 10000
N = 10000
E = 320000
D = 128

NC = 2
NS = 16
NW = NC * NS
EPW = E // NW
CB = 64
NCHUNK = E // CB
NQMAX = (NCHUNK + NW - 1) // NW
ZCH = U // CB
ZTAIL = U - ZCH * CB
CNTB = 200
OCB = 200
NOCH = U // OCB

DEPTH = 3
NGROUP = (NQMAX + DEPTH - 1) // DEPTH


def _sc_pass_body(table, gidx, sidx, edge, acc_out, cnt_out, *refs):
    gidx_v = refs[0:DEPTH]
    sidx_v = refs[DEPTH:2 * DEPTH]
    edge_v = refs[2 * DEPTH:3 * DEPTH]
    rows_v = refs[3 * DEPTH:4 * DEPTH]
    zrow_v, ones_v, cnt_v = refs[4 * DEPTH:4 * DEPTH + 3]
    loadsem = refs[4 * DEPTH + 3:5 * DEPTH + 3]
    gathsem = refs[5 * DEPTH + 3:6 * DEPTH + 3]
    scatsem = refs[6 * DEPTH + 3:7 * DEPTH + 3]
    acc_sp, cnt_sp = refs[7 * DEPTH + 3:7 * DEPTH + 5]

    c = lax.axis_index("c")
    s = lax.axis_index("s")
    wid = s * NC + c
    if NCHUNK % NW:
        nq = jnp.where(wid < NCHUNK % NW, NQMAX, NQMAX - 1).astype(jnp.int32)
    else:
        nq = jnp.int32(NCHUNK // NW)

    def _fill_small(i, carry):
        zrow_v[pl.ds(i * 16, 16)] = jnp.zeros((16,), jnp.float32)
        ones_v[pl.ds(i * 16, 16)] = jnp.full((16,), 1.0, jnp.float32)
        return carry
    lax.fori_loop(0, CB // 16, _fill_small, 0)

    def _fill_zbuf(r, carry):
        for j in range(D // 16):
            edge_v[0][r, pl.ds(j * 16, 16)] = jnp.zeros((16,), jnp.float32)
        return carry
    lax.fori_loop(0, CB, _fill_zbuf, 0)

    def _zero(k, carry):
        ch = s + k * NS

        @pl.when(ch < ZCH)
        def _():
            pltpu.sync_copy(edge_v[0], acc_sp.at[pl.ds(ch * CB, CB)])
            pltpu.sync_copy(zrow_v, cnt_sp.at[pl.ds(ch * CB, CB)])
        return carry
    lax.fori_loop(0, (ZCH + NS - 1) // NS, _zero, 0)

    @pl.when(s == 0)
    def _():
        pltpu.sync_copy(edge_v[0].at[pl.ds(0, ZTAIL)],
                        acc_sp.at[pl.ds(ZCH * CB, ZTAIL)])
        pltpu.sync_copy(zrow_v.at[pl.ds(0, ZTAIL)],
                        cnt_sp.at[pl.ds(ZCH * CB, ZTAIL)])

    plsc.subcore_barrier()

    def _start_loads(q, j):
        base = (q * NW + wid) * CB
        pltpu.async_copy(gidx.at[pl.ds(base, CB)], gidx_v[j], loadsem[j])
        pltpu.async_copy(sidx.at[pl.ds(base, CB)], sidx_v[j], loadsem[j])
        pltpu.async_copy(edge.at[pl.ds(base, CB)], edge_v[j], loadsem[j])

    def _wait_loads(j):
        pltpu.make_async_copy(gidx.at[pl.ds(0, CB)], gidx_v[j], loadsem[j]).wait()
        pltpu.make_async_copy(sidx.at[pl.ds(0, CB)], sidx_v[j], loadsem[j]).wait()
        pltpu.make_async_copy(edge.at[pl.ds(0, CB)], edge_v[j], loadsem[j]).wait()

    def _start_gather(j):
        pltpu.async_copy(table.at[gidx_v[j]], rows_v[j], gathsem[j])

    def _wait_gather(j):
        pltpu.make_async_copy(table.at[gidx_v[j]], rows_v[j], gathsem[j]).wait()

    def _start_scatter(j):
        pltpu.async_copy(edge_v[j], acc_sp.at[sidx_v[j]], scatsem[j], add=True)
        pltpu.async_copy(ones_v, cnt_sp.at[sidx_v[j]], scatsem[j], add=True)

    def _wait_scatter(j):
        pltpu.make_async_copy(edge_v[j], acc_sp.at[sidx_v[j]], scatsem[j]).wait()
        pltpu.make_async_copy(ones_v, cnt_sp.at[sidx_v[j]], scatsem[j]).wait()

    for q0 in range(DEPTH - 1):
        _start_loads(q0, q0)
    _wait_loads(0)
    _start_gather(0)

    def _group(g, carry):
        for j in range(DEPTH):
            q = g * DEPTH + j
            j1 = (j + 1) % DEPTH
            jp = (j - 1) % DEPTH

            @pl.when(q < nq)
            def _():
                @pl.when(q + 1 < nq)
                def _():
                    _wait_loads(j1)
                    _start_gather(j1)

                @pl.when(q >= 1)
                def _():
                    _wait_scatter(jp)

                @pl.when(q + DEPTH - 1 < nq)
                def _():
                    _start_loads(q + DEPTH - 1, jp)

                _wait_gather(j)

                def _mul(r, carry2):
                    for jj in range(D // 16):
                        sl = pl.ds(jj * 16, 16)
                        edge_v[j][r, sl] = rows_v[j][r, sl] * edge_v[j][r, sl]
                    return carry2
                lax.fori_loop(0, CB, _mul, 0)

                _start_scatter(j)
        return carry
    lax.fori_loop(0, NGROUP, _group, 0)

    for jd in range(DEPTH):
        @pl.when(lax.rem(nq - 1, jnp.int32(DEPTH)) == jd)
        def _(jd=jd):
            _wait_scatter(jd)

    plsc.subcore_barrier()

    def _copy_out(k, carry):
        ch = s + k * NS

        @pl.when(ch < NOCH)
        def _():
            pltpu.sync_copy(acc_sp.at[pl.ds(ch * OCB, OCB)],
                            acc_out.at[pl.ds(c * U + ch * OCB, OCB)])
        return carry
    lax.fori_loop(0, (NOCH + NS - 1) // NS, _copy_out, 0)

    def _cnt_out(k, carry):
        ch = s + k * NS

        @pl.when(ch < U // CNTB)
        def _():
            pltpu.sync_copy(cnt_sp.at[pl.ds(ch * CNTB, CNTB)], cnt_v)
            pltpu.sync_copy(cnt_v, cnt_out.at[pl.ds(c * U + ch * CNTB, CNTB)])
        return carry
    lax.fori_loop(0, (U // CNTB + NS - 1) // NS, _cnt_out, 0)


_sc_pass = functools.partial(
    pl.kernel,
    out_type=(jax.ShapeDtypeStruct((NC * U, D), jnp.float32),
              jax.ShapeDtypeStruct((NC * U,), jnp.float32)),
    mesh=plsc.VectorSubcoreMesh(core_axis_name="c", subcore_axis_name="s",
                                num_cores=NC, num_subcores=NS),
    scratch_types=(
        [pltpu.VMEM((CB,), jnp.int32) for _ in range(DEPTH)]
        + [pltpu.VMEM((CB,), jnp.int32) for _ in range(DEPTH)]
        + [pltpu.VMEM((CB, D), jnp.float32) for _ in range(DEPTH)]
        + [pltpu.VMEM((CB, D), jnp.float32) for _ in range(DEPTH)]
        + [pltpu.VMEM((CB,), jnp.float32),
           pltpu.VMEM((CB,), jnp.float32),
           pltpu.VMEM((CNTB,), jnp.float32)]
        + [pltpu.SemaphoreType.DMA for _ in range(3 * DEPTH)]
        + [pltpu.VMEM_SHARED((U, D), jnp.float32),
           pltpu.VMEM_SHARED((U,), jnp.float32)]
    ),
)(_sc_pass_body)


BU = 1000


def _lstm_body(unew_ref, hn_ref, cs_ref, wih_ref, whh_ref,
               b_ref, hn_out_ref, cs_out_ref):
    gates = (jnp.dot(unew_ref[...], wih_ref[...],
                     preferred_element_type=jnp.float32)
             + jnp.dot(hn_ref[...], whh_ref[...],
                       preferred_element_type=jnp.float32)
             + b_ref[...])
    i = jax.nn.sigmoid(gates[:, 0:D])
    f = jax.nn.sigmoid(gates[:, D:2 * D])
    g = jnp.tanh(gates[:, 2 * D:3 * D])
    o = jax.nn.sigmoid(gates[:, 3 * D:4 * D])
    cs = f * cs_ref[...] + i * g
    cs_out_ref[...] = cs
    hn_out_ref[...] = o * jnp.tanh(cs)


def _norm_body(acc_ref, cnt_ref, nf_ref, out_ref):
    a = acc_ref[0] + acc_ref[1]
    cnt = cnt_ref[0] + cnt_ref[1]
    out_ref[...] = a / jnp.maximum(cnt, 1.0) + nf_ref[...]


def kernel(user_feat, news_feat, edge_src_user, edge_dst_news, edge_feat,
           edge_feat_rev, prev_hn, prev_cs, W_ih, W_hh, b_ih, b_hh):
    gidx1 = edge_dst_news.astype(jnp.int32)
    sidx1 = edge_src_user.astype(jnp.int32)

    acc1, cnt1 = _sc_pass(news_feat, gidx1, sidx1, edge_feat_rev)
    acc1 = acc1.reshape(NC, U, D)
    cnt1 = cnt1.reshape(NC, U, 1)

    wihT = W_ih.T
    whhT = W_hh.T
    b = (b_ih + b_hh).reshape(1, 4 * D)

    user_new = pl.pallas_call(
        _norm_body,
        grid=(U // BU,),
        in_specs=[
            pl.BlockSpec((NC, BU, D), lambda i: (0, i, 0)),
            pl.BlockSpec((NC, BU, 1), lambda i: (0, i, 0)),
            pl.BlockSpec((BU, D), lambda i: (i, 0)),
        ],
        out_specs=pl.BlockSpec((BU, D), lambda i: (i, 0)),
        out_shape=jax.ShapeDtypeStruct((U, D), jnp.float32),
    )(acc1, cnt1, user_feat)

    user_hn, user_cs = pl.pallas_call(
        _lstm_body,
        grid=(U // BU,),
        in_specs=[
            pl.BlockSpec((BU, D), lambda i: (i, 0)),
            pl.BlockSpec((BU, D), lambda i: (i, 0)),
            pl.BlockSpec((BU, D), lambda i: (i, 0)),
            pl.BlockSpec((D, 4 * D), lambda i: (0, 0)),
            pl.BlockSpec((D, 4 * D), lambda i: (0, 0)),
            pl.BlockSpec((1, 4 * D), lambda i: (0, 0)),
        ],
        out_specs=[
            pl.BlockSpec((BU, D), lambda i: (i, 0)),
            pl.BlockSpec((BU, D), lambda i: (i, 0)),
        ],
        out_shape=[
            jax.ShapeDtypeStruct((U, D), jnp.float32),
            jax.ShapeDtypeStruct((U, D), jnp.float32),
        ],
    )(user_new, prev_hn, prev_cs, wihT, whhT, b)

    acc2, cnt2 = _sc_pass(user_new, sidx1, gidx1, edge_feat)
    acc2 = acc2.reshape(NC, N, D)
    cnt2 = cnt2.reshape(NC, N, 1)

    news_new = pl.pallas_call(
        _norm_body,
        grid=(N // BU,),
        in_specs=[
            pl.BlockSpec((NC, BU, D), lambda i: (0, i, 0)),
            pl.BlockSpec((NC, BU, 1), lambda i: (0, i, 0)),
            pl.BlockSpec((BU, D), lambda i: (i, 0)),
        ],
        out_specs=pl.BlockSpec((BU, D), lambda i: (i, 0)),
        out_shape=jax.ShapeDtypeStruct((N, D), jnp.float32),
    )(acc2, cnt2, news_feat)

    return user_hn, user_cs, news_new
